# baseline (speedup 1.0000x reference)
import functools

import jax
import jax.numpy as jnp
from jax import lax
from jax.experimental import pallas as pl
from jax.experimental.pallas import tpu as pltpu
from jax.experimental.pallas import tpu_sc as plsc

ALPHA = 0.7
TAU_WORD = 0.1

NC = 2
NS = 16
L = 16
NW = NC * NS
UNROLL = 8
LANE_TILE = 128


@functools.lru_cache(maxsize=None)
def _make_sc_call(B, T, V):
    N = B * T
    v_main = (V // LANE_TILE) * LANE_TILE
    v_tail = V - v_main
    assert N % (2 * NW) == 0 and v_main % (L * UNROLL) == 0
    assert v_tail == L
    tok_per_w = N // NW
    n_chunk = v_main // (L * UNROLL)
    mesh = plsc.VectorSubcoreMesh(core_axis_name="c", subcore_axis_name="s")

    @functools.partial(
        pl.kernel,
        out_type=jax.ShapeDtypeStruct((NW, 4 * L), jnp.float32),
        mesh=mesh,
        compiler_params=pltpu.CompilerParams(needs_layout_passes=False),
        scratch_types=[
            pltpu.VMEM((8 * N,), jnp.int32),
            pltpu.VMEM((N,), jnp.float32),
            pltpu.VMEM((1, V), jnp.float32),
            pltpu.VMEM((1, V), jnp.float32),
            pltpu.VMEM((1, v_main), jnp.float32),
            pltpu.VMEM((1, v_main), jnp.float32),
            pltpu.VMEM((1, LANE_TILE), jnp.float32),
            pltpu.VMEM((1, LANE_TILE), jnp.float32),
            pltpu.VMEM((4 * L,), jnp.float32),
            pltpu.SemaphoreType.DMA,
            pltpu.SemaphoreType.DMA,
        ],
    )
    def sc_call(in_hbm, tgt_hbm, msk_hbm, sim_hbm, tail_hbm, out_hbm,
                tgt_v, msk_v, in_v0, in_v1, sim_v0, sim_v1, tl_v0, tl_v1,
                res_v, sem0, sem1):
        wid = lax.axis_index("s") * NC + lax.axis_index("c")
        base = wid * tok_per_w
        pltpu.sync_copy(tgt_hbm, tgt_v)
        pltpu.sync_copy(msk_hbm, msk_v)
        lane = lax.iota(jnp.int32, L)
        zeros_i = jnp.zeros((L,), jnp.int32)

        def fire(t, in_buf, sim_buf, tl_buf, sem):
            bi = t // T
            pltpu.async_copy(
                in_hbm.at[bi, pl.ds(t - bi * T, 1)], in_buf, sem)
            idx = tgt_v.at[pl.ds(8 * t, 1)]
            pltpu.async_copy(sim_hbm.at[idx, pl.ds(0, v_main)], sim_buf, sem)
            pltpu.async_copy(tail_hbm.at[idx], tl_buf, sem)

        def wait(t, in_buf, sim_buf, tl_buf, sem):
            bi = t // T
            idx = tgt_v.at[pl.ds(8 * t, 1)]
            pltpu.make_async_copy(
                in_hbm.at[bi, pl.ds(t - bi * T, 1)], in_buf, sem).wait()
            pltpu.make_async_copy(
                sim_hbm.at[idx, pl.ds(0, v_main)], sim_buf, sem).wait()
            pltpu.make_async_copy(tail_hbm.at[idx], tl_buf, sem).wait()

        def compute(t, in_buf, sim_buf, tl_buf, accs):
            acc_n, acc_d, acc_ml, acc_m = accs

            def inner(k, c):
                ns, ds = c
                off = k * (L * UNROLL)
                ns_out, ds_out = [], []
                for u in range(UNROLL):
                    vs = sim_buf[0, pl.ds(off + u * L, L)]
                    vi = in_buf[0, pl.ds(off + u * L, L)]
                    e = jnp.exp(vs * (1.0 / TAU_WORD) - (1.0 / TAU_WORD))
                    ns_out.append(ns[u] + vi * e)
                    ds_out.append(ds[u] + e)
                return (tuple(ns_out), tuple(ds_out))

            zf = jnp.zeros((L,), jnp.float32)
            zs = (zf,) * UNROLL
            ns, ds = lax.fori_loop(0, n_chunk, inner, (zs, zs))
            tn = functools.reduce(lambda a, b: a + b, ns)
            td = functools.reduce(lambda a, b: a + b, ds)
            vs = tl_buf[0, pl.ds(LANE_TILE - L, L)]
            vi = in_buf[0, pl.ds(v_main, L)]
            e = jnp.exp(vs * (1.0 / TAU_WORD) - (1.0 / TAU_WORD))
            tn = tn + vi * e
            td = td + e
            tv = jnp.full((L,), t, jnp.int32)
            mv = plsc.load_gather(msk_v, [tv])
            tgt_vec = plsc.load_gather(tgt_v, [tv * 8])
            g = plsc.load_gather(in_buf, [zeros_i, tgt_vec])
            lane0 = lane == 0
            return (acc_n + tn * mv,
                    acc_d + td * mv,
                    acc_ml + jnp.where(lane0, g * mv, 0.0),
                    acc_m + jnp.where(lane0, mv, 0.0))

        zf = jnp.zeros((L,), jnp.float32)
        fire(base, in_v0, sim_v0, tl_v0, sem0)

        def pair(k, accs):
            t0 = base + 2 * k
            t1 = t0 + 1
            fire(t1, in_v1, sim_v1, tl_v1, sem1)
            wait(t0, in_v0, sim_v0, tl_v0, sem0)
            accs = compute(t0, in_v0, sim_v0, tl_v0, accs)

            @pl.when(2 * k + 2 < tok_per_w)
            def _():
                fire(t0 + 2, in_v0, sim_v0, tl_v0, sem0)

            wait(t1, in_v1, sim_v1, tl_v1, sem1)
            accs = compute(t1, in_v1, sim_v1, tl_v1, accs)
            return accs

        acc_n, acc_d, acc_ml, acc_m = lax.fori_loop(
            0, tok_per_w // 2, pair, (zf, zf, zf, zf))
        res_v[pl.ds(0, L)] = acc_n
        res_v[pl.ds(L, L)] = acc_d
        res_v[pl.ds(2 * L, L)] = acc_ml
        res_v[pl.ds(3 * L, L)] = acc_m
        pltpu.sync_copy(res_v, out_hbm.at[wid])

    return sc_call


def kernel(input, target, mask, Sim_Matrix):
    b, t, v = input.shape
    flat_t = target[:, :t].reshape(-1).astype(jnp.int32)
    n = flat_t.shape[0]
    tgt8 = jnp.broadcast_to(flat_t[:, None], (n, 8)).reshape(-1)
    flat_m = mask[:, :t].reshape(-1).astype(jnp.float32)
    sim_tail = Sim_Matrix[:, v - LANE_TILE:]
    partials = _make_sc_call(b, t, v)(
        input, tgt8, flat_m, Sim_Matrix, sim_tail)
    p = partials.reshape(NW, 4, L)
    num = jnp.sum(p[:, 0, :])
    den = jnp.sum(p[:, 1, :])
    ml_sum = jnp.sum(p[:, 2, :])
    m_sum = jnp.sum(p[:, 3, :])
    ml_output = -ml_sum / m_sum
    smooth_loss = -num / den
    total = ALPHA * smooth_loss + (1.0 - ALPHA) * ml_output
    return (ml_output, total)


# speedup vs baseline: 2.0942x; 2.0942x over previous
import functools

import jax
import jax.numpy as jnp
from jax import lax
from jax.experimental import pallas as pl
from jax.experimental.pallas import tpu as pltpu
from jax.experimental.pallas import tpu_sc as plsc

ALPHA = 0.7
TAU_WORD = 0.1

NC = 2
NS = 16
L = 16
NW = NC * NS
UNROLL = 13
LANE_TILE = 128
TC_TOKENS = 1152
TC_BLK = 16


@functools.lru_cache(maxsize=None)
def _make_sc_call(B, T, V, K):
    N = B * T
    n_sc = N - K
    v_main = (V // LANE_TILE) * LANE_TILE
    v_tail = V - v_main
    assert n_sc % (8 * NW) == 0 and v_main % (L * UNROLL) == 0
    assert v_tail == L and B % 2 == 0 and K % 8 == 0
    tok_per_w = n_sc // NW
    n_pairs = tok_per_w // 2
    n_chunk = v_main // (L * UNROLL)
    mesh = plsc.VectorSubcoreMesh(core_axis_name="c", subcore_axis_name="s")

    @functools.partial(
        pl.kernel,
        out_type=jax.ShapeDtypeStruct((NW, 4 * L), jnp.float32),
        mesh=mesh,
        compiler_params=pltpu.CompilerParams(needs_layout_passes=False),
        scratch_types=[
            pltpu.VMEM((8 * tok_per_w,), jnp.int32),
            pltpu.VMEM((tok_per_w,), jnp.float32),
            pltpu.VMEM((2, V), jnp.float32),
            pltpu.VMEM((2, V), jnp.float32),
            pltpu.VMEM((2, v_main), jnp.float32),
            pltpu.VMEM((2, v_main), jnp.float32),
            pltpu.VMEM((2, LANE_TILE), jnp.float32),
            pltpu.VMEM((2, LANE_TILE), jnp.float32),
            pltpu.VMEM((4 * L,), jnp.float32),
            pltpu.SemaphoreType.DMA,
            pltpu.SemaphoreType.DMA,
        ],
    )
    def sc_call(in_hbm, tgt_hbm, msk_hbm, sim_hbm, tail_hbm, out_hbm,
                tgt_v, msk_v, in_v0, in_v1, sim_v0, sim_v1, tl_v0, tl_v1,
                res_v, sem0, sem1):
        wid = lax.axis_index("s") * NC + lax.axis_index("c")
        base = K + wid * tok_per_w
        pltpu.sync_copy(
            tgt_hbm.at[pl.ds(8 * base, 8 * tok_per_w)], tgt_v)
        pltpu.sync_copy(
            msk_hbm.at[pl.ds(base, tok_per_w)], msk_v)
        lane = lax.iota(jnp.int32, L)

        def fire_pair(p, in_buf, sim_buf, tl_buf, sem):
            t0 = base + 2 * p
            ti = t0 // B
            pltpu.async_copy(
                in_hbm.at[ti, pl.ds(t0 - ti * B, 2)], in_buf, sem)
            idx = tgt_v.at[pl.ds(16 * p, 2)]
            pltpu.async_copy(sim_hbm.at[idx, pl.ds(0, v_main)], sim_buf, sem)
            pltpu.async_copy(tail_hbm.at[idx], tl_buf, sem)

        def wait_pair(p, in_buf, sim_buf, tl_buf, sem):
            t0 = base + 2 * p
            ti = t0 // B
            idx = tgt_v.at[pl.ds(16 * p, 2)]
            pltpu.make_async_copy(
                in_hbm.at[ti, pl.ds(t0 - ti * B, 2)], in_buf, sem).wait()
            pltpu.make_async_copy(
                sim_hbm.at[idx, pl.ds(0, v_main)], sim_buf, sem).wait()
            pltpu.make_async_copy(tail_hbm.at[idx], tl_buf, sem).wait()

        def compute(p, row, in_buf, sim_buf, tl_buf, accs):
            acc_n, acc_d, acc_ml, acc_m = accs
            j = 2 * p + row

            def inner(k, c):
                ns, ds = c
                off = k * (L * UNROLL)
                ns_out, ds_out = [], []
                for u in range(UNROLL):
                    vs = sim_buf[row, pl.ds(off + u * L, L)]
                    vi = in_buf[row, pl.ds(off + u * L, L)]
                    e = jnp.exp(vs * (1.0 / TAU_WORD))
                    ns_out.append(ns[u] + vi * e)
                    ds_out.append(ds[u] + e)
                return (tuple(ns_out), tuple(ds_out))

            zf = jnp.zeros((L,), jnp.float32)
            zs = (zf,) * UNROLL
            ns, ds = lax.fori_loop(0, n_chunk, inner, (zs, zs))
            tn = functools.reduce(lambda a, b: a + b, ns)
            td = functools.reduce(lambda a, b: a + b, ds)
            vs = tl_buf[row, pl.ds(LANE_TILE - L, L)]
            vi = in_buf[row, pl.ds(v_main, L)]
            e = jnp.exp(vs * (1.0 / TAU_WORD))
            tn = tn + vi * e
            td = td + e
            mv = plsc.load_gather(msk_v, [jnp.full((L,), j, jnp.int32)])
            tgt_vec = plsc.load_gather(
                tgt_v, [jnp.full((L,), 16 * p + row, jnp.int32)])
            g = plsc.load_gather(
                in_buf, [jnp.full((L,), row, jnp.int32), tgt_vec])
            lane0 = lane == 0
            mvs = mv * 4.5399929762484854e-05
            return (acc_n + tn * mvs,
                    acc_d + td * mvs,
                    acc_ml + jnp.where(lane0, g * mv, 0.0),
                    acc_m + jnp.where(lane0, mv, 0.0))

        def compute_pair(p, in_buf, sim_buf, tl_buf, accs):
            accs = compute(p, 0, in_buf, sim_buf, tl_buf, accs)
            return compute(p, 1, in_buf, sim_buf, tl_buf, accs)

        zf = jnp.zeros((L,), jnp.float32)
        fire_pair(0, in_v0, sim_v0, tl_v0, sem0)

        def quad(q, accs):
            p0 = 2 * q
            p1 = p0 + 1
            fire_pair(p1, in_v1, sim_v1, tl_v1, sem1)
            wait_pair(p0, in_v0, sim_v0, tl_v0, sem0)
            accs = compute_pair(p0, in_v0, sim_v0, tl_v0, accs)

            @pl.when(p0 + 2 < n_pairs)
            def _():
                fire_pair(p0 + 2, in_v0, sim_v0, tl_v0, sem0)

            wait_pair(p1, in_v1, sim_v1, tl_v1, sem1)
            accs = compute_pair(p1, in_v1, sim_v1, tl_v1, accs)
            return accs

        acc_n, acc_d, acc_ml, acc_m = lax.fori_loop(
            0, n_pairs // 2, quad, (zf, zf, zf, zf))
        res_v[pl.ds(0, L)] = acc_n
        res_v[pl.ds(L, L)] = acc_d
        res_v[pl.ds(2 * L, L)] = acc_ml
        res_v[pl.ds(3 * L, L)] = acc_m
        pltpu.sync_copy(res_v, out_hbm.at[wid])

    return sc_call


@functools.lru_cache(maxsize=None)
def _make_tc_call(B, T, V, K):
    assert K % TC_BLK == 0 and B % TC_BLK == 0
    steps = K // TC_BLK
    bb = B // TC_BLK

    def body(tgt_s, msk_s, in_ref, sim_any, out_num, out_den, out_ml, out_m,
             sim_buf, sems):
        j = pl.program_id(0)

        def fire(step, slot):
            t0 = step * TC_BLK
            for u in range(TC_BLK):
                pltpu.make_async_copy(
                    sim_any.at[pl.ds(tgt_s[t0 + u], 1)],
                    sim_buf.at[slot, pl.ds(u, 1)],
                    sems.at[slot],
                ).start()

        def drain(step, slot):
            t0 = step * TC_BLK
            for u in range(TC_BLK):
                pltpu.make_async_copy(
                    sim_any.at[pl.ds(tgt_s[t0 + u], 1)],
                    sim_buf.at[slot, pl.ds(u, 1)],
                    sems.at[slot],
                ).wait()

        @pl.when(j == 0)
        def _():
            out_num[0, 0] = 0.0
            out_den[0, 0] = 0.0
            out_ml[0, 0] = 0.0
            out_m[0, 0] = 0.0
            fire(0, 0)
            if steps > 1:
                fire(1, 1)
            if steps > 2:
                fire(2, 2)

        @pl.when(j + 3 < steps)
        def _():
            fire(j + 3, (j + 3) % 4)

        slot = j % 4
        drain(j, slot)
        sim8 = sim_buf[slot]
        c2 = 1.4426950408889634 / TAU_WORD
        e = jnp.exp2(sim8 * c2 - c2)
        in8 = in_ref[0]
        t0 = j * TC_BLK
        mcol = jnp.stack(
            [msk_s[t0 + u] for u in range(TC_BLK)]).reshape(TC_BLK, 1)
        tcol = jnp.stack(
            [tgt_s[t0 + u] for u in range(TC_BLK)]).reshape(TC_BLK, 1)
        s_num = jnp.sum(in8 * e, axis=1, keepdims=True)
        s_den = jnp.sum(e, axis=1, keepdims=True)
        lane = lax.broadcasted_iota(jnp.int32, (TC_BLK, V), 1)
        g = jnp.sum(jnp.where(lane == tcol, in8, 0.0), axis=1, keepdims=True)
        out_num[0, 0] += jnp.sum(s_num * mcol)
        out_den[0, 0] += jnp.sum(s_den * mcol)
        out_ml[0, 0] += jnp.sum(g * mcol)
        out_m[0, 0] += jnp.sum(mcol)

    in_spec = pl.BlockSpec(
        (1, TC_BLK, V), lambda j, tgt, msk: (j // bb, j % bb, 0))
    sim_spec = pl.BlockSpec(memory_space=pl.ANY)
    scalar_spec = pl.BlockSpec(memory_space=pltpu.SMEM)
    grid_spec = pltpu.PrefetchScalarGridSpec(
        num_scalar_prefetch=2,
        grid=(steps,),
        in_specs=[in_spec, sim_spec],
        out_specs=[scalar_spec] * 4,
        scratch_shapes=[
            pltpu.VMEM((4, TC_BLK, V), jnp.float32),
            pltpu.SemaphoreType.DMA((4,)),
        ],
    )
    return pl.pallas_call(
        body,
        grid_spec=grid_spec,
        out_shape=[jax.ShapeDtypeStruct((1, 1), jnp.float32)] * 4,
        compiler_params=pltpu.CompilerParams(
            dimension_semantics=("arbitrary",)),
    )


def _epilogue_body(p_ref, tn_ref, td_ref, tml_ref, tm_ref, oml_ref, otot_ref):
    p = p_ref[...]
    num = jnp.sum(p[:, 0:L]) + tn_ref[0, 0]
    den = jnp.sum(p[:, L:2 * L]) + td_ref[0, 0]
    ml_sum = jnp.sum(p[:, 2 * L:3 * L]) + tml_ref[0, 0]
    m_sum = jnp.sum(p[:, 3 * L:4 * L]) + tm_ref[0, 0]
    ml_output = -ml_sum / m_sum
    total = ALPHA * (-num / den) + (1.0 - ALPHA) * ml_output
    oml_ref[0, 0] = ml_output
    otot_ref[0, 0] = total


_epilogue = pl.pallas_call(
    _epilogue_body,
    out_shape=[jax.ShapeDtypeStruct((1, 1), jnp.float32)] * 2,
    out_specs=[pl.BlockSpec(memory_space=pltpu.SMEM)] * 2,
)

def kernel(input, target, mask, Sim_Matrix):
    b, t, v = input.shape
    input_t = jnp.swapaxes(input, 0, 1)
    flat_t = jnp.swapaxes(target[:, :t], 0, 1).reshape(-1).astype(jnp.int32)
    n = flat_t.shape[0]
    tgt8 = jnp.pad(flat_t.reshape(n // 2, 2),
                   ((0, 0), (0, 14))).reshape(-1)
    flat_m = jnp.swapaxes(mask[:, :t], 0, 1).reshape(-1).astype(jnp.float32)
    sim_tail = Sim_Matrix[:, v - LANE_TILE:]
    k = TC_TOKENS if (n - TC_TOKENS) % (8 * NW) == 0 and b % TC_BLK == 0 else 0
    partials = _make_sc_call(b, t, v, k)(
        input_t, tgt8, flat_m, Sim_Matrix, sim_tail)
    if k:
        tc_num, tc_den, tc_ml, tc_m = _make_tc_call(b, t, v, k)(
            flat_t, flat_m, input_t, Sim_Matrix)
    else:
        z = jnp.zeros((1, 1), jnp.float32)
        tc_num = tc_den = tc_ml = tc_m = z
    oml, otot = _epilogue(partials, tc_num, tc_den, tc_ml, tc_m)
    return (oml.reshape(()), otot.reshape(()))


# speedup vs baseline: 2.0986x; 1.0021x over previous
import functools

import jax
import jax.numpy as jnp
from jax import lax
from jax.experimental import pallas as pl
from jax.experimental.pallas import tpu as pltpu
from jax.experimental.pallas import tpu_sc as plsc

ALPHA = 0.7
TAU_WORD = 0.1

NC = 2
NS = 16
L = 16
NW = NC * NS
UNROLL = 8
LANE_TILE = 128
TC_TOKENS = 1152
TC_BLK = 16


@functools.lru_cache(maxsize=None)
def _make_sc_call(B, T, V, K):
    N = B * T
    n_sc = N - K
    v_main = (V // LANE_TILE) * LANE_TILE
    v_tail = V - v_main
    assert n_sc % (8 * NW) == 0 and v_main % (L * UNROLL) == 0
    assert v_tail == L and B % 2 == 0 and K % 8 == 0
    tok_per_w = n_sc // NW
    n_pairs = tok_per_w // 2
    n_chunk = v_main // (L * UNROLL)
    mesh = plsc.VectorSubcoreMesh(core_axis_name="c", subcore_axis_name="s")

    @functools.partial(
        pl.kernel,
        out_type=jax.ShapeDtypeStruct((NW, 4 * L), jnp.float32),
        mesh=mesh,
        compiler_params=pltpu.CompilerParams(needs_layout_passes=False),
        scratch_types=[
            pltpu.VMEM((8 * tok_per_w,), jnp.int32),
            pltpu.VMEM((tok_per_w,), jnp.float32),
            pltpu.VMEM((2, V), jnp.float32),
            pltpu.VMEM((2, V), jnp.float32),
            pltpu.VMEM((2, v_main), jnp.float32),
            pltpu.VMEM((2, v_main), jnp.float32),
            pltpu.VMEM((2, LANE_TILE), jnp.float32),
            pltpu.VMEM((2, LANE_TILE), jnp.float32),
            pltpu.VMEM((4 * L,), jnp.float32),
            pltpu.SemaphoreType.DMA,
            pltpu.SemaphoreType.DMA,
        ],
    )
    def sc_call(in_hbm, tgt_hbm, msk_hbm, sim_hbm, tail_hbm, out_hbm,
                tgt_v, msk_v, in_v0, in_v1, sim_v0, sim_v1, tl_v0, tl_v1,
                res_v, sem0, sem1):
        wid = lax.axis_index("s") * NC + lax.axis_index("c")
        base = K + wid * tok_per_w
        pltpu.sync_copy(
            tgt_hbm.at[pl.ds(8 * base, 8 * tok_per_w)], tgt_v)
        pltpu.sync_copy(
            msk_hbm.at[pl.ds(base, tok_per_w)], msk_v)
        lane = lax.iota(jnp.int32, L)

        def fire_pair(p, in_buf, sim_buf, tl_buf, sem):
            t0 = base + 2 * p
            ti = t0 // B
            pltpu.async_copy(
                in_hbm.at[ti, pl.ds(t0 - ti * B, 2)], in_buf, sem)
            idx = tgt_v.at[pl.ds(16 * p, 2)]
            pltpu.async_copy(sim_hbm.at[idx, pl.ds(0, v_main)], sim_buf, sem)
            pltpu.async_copy(tail_hbm.at[idx], tl_buf, sem)

        def wait_pair(p, in_buf, sim_buf, tl_buf, sem):
            t0 = base + 2 * p
            ti = t0 // B
            idx = tgt_v.at[pl.ds(16 * p, 2)]
            pltpu.make_async_copy(
                in_hbm.at[ti, pl.ds(t0 - ti * B, 2)], in_buf, sem).wait()
            pltpu.make_async_copy(
                sim_hbm.at[idx, pl.ds(0, v_main)], sim_buf, sem).wait()
            pltpu.make_async_copy(tail_hbm.at[idx], tl_buf, sem).wait()

        def compute(p, row, in_buf, sim_buf, tl_buf, accs):
            acc_n, acc_d, acc_ml, acc_m = accs
            j = 2 * p + row

            def inner(k, c):
                ns, ds = c
                off = k * (L * UNROLL)
                ns_out, ds_out = [], []
                for u in range(UNROLL):
                    vs = sim_buf[row, pl.ds(off + u * L, L)]
                    vi = in_buf[row, pl.ds(off + u * L, L)]
                    e = jnp.exp(vs * (1.0 / TAU_WORD))
                    ns_out.append(ns[u] + vi * e)
                    ds_out.append(ds[u] + e)
                return (tuple(ns_out), tuple(ds_out))

            zf = jnp.zeros((L,), jnp.float32)
            zs = (zf,) * UNROLL
            ns, ds = lax.fori_loop(0, n_chunk, inner, (zs, zs))
            tn = functools.reduce(lambda a, b: a + b, ns)
            td = functools.reduce(lambda a, b: a + b, ds)
            vs = tl_buf[row, pl.ds(LANE_TILE - L, L)]
            vi = in_buf[row, pl.ds(v_main, L)]
            e = jnp.exp(vs * (1.0 / TAU_WORD))
            tn = tn + vi * e
            td = td + e
            mv = plsc.load_gather(msk_v, [jnp.full((L,), j, jnp.int32)])
            tgt_vec = plsc.load_gather(
                tgt_v, [jnp.full((L,), 16 * p + row, jnp.int32)])
            g = plsc.load_gather(
                in_buf, [jnp.full((L,), row, jnp.int32), tgt_vec])
            lane0 = lane == 0
            mvs = mv * 4.5399929762484854e-05
            return (acc_n + tn * mvs,
                    acc_d + td * mvs,
                    acc_ml + jnp.where(lane0, g * mv, 0.0),
                    acc_m + jnp.where(lane0, mv, 0.0))

        def compute_pair(p, in_buf, sim_buf, tl_buf, accs):
            accs = compute(p, 0, in_buf, sim_buf, tl_buf, accs)
            return compute(p, 1, in_buf, sim_buf, tl_buf, accs)

        zf = jnp.zeros((L,), jnp.float32)
        fire_pair(0, in_v0, sim_v0, tl_v0, sem0)

        def quad(q, accs):
            p0 = 2 * q
            p1 = p0 + 1
            fire_pair(p1, in_v1, sim_v1, tl_v1, sem1)
            wait_pair(p0, in_v0, sim_v0, tl_v0, sem0)
            accs = compute_pair(p0, in_v0, sim_v0, tl_v0, accs)

            @pl.when(p0 + 2 < n_pairs)
            def _():
                fire_pair(p0 + 2, in_v0, sim_v0, tl_v0, sem0)

            wait_pair(p1, in_v1, sim_v1, tl_v1, sem1)
            accs = compute_pair(p1, in_v1, sim_v1, tl_v1, accs)
            return accs

        acc_n, acc_d, acc_ml, acc_m = lax.fori_loop(
            0, n_pairs // 2, quad, (zf, zf, zf, zf))
        res_v[pl.ds(0, L)] = acc_n
        res_v[pl.ds(L, L)] = acc_d
        res_v[pl.ds(2 * L, L)] = acc_ml
        res_v[pl.ds(3 * L, L)] = acc_m
        pltpu.sync_copy(res_v, out_hbm.at[wid])

    return sc_call


@functools.lru_cache(maxsize=None)
def _make_tc_call(B, T, V, K):
    assert K % TC_BLK == 0 and B % TC_BLK == 0
    steps = K // TC_BLK
    bb = B // TC_BLK

    def body(tgt_s, msk_s, in_ref, sim_any, out_num, out_den, out_ml, out_m,
             sim_buf, sems):
        j = pl.program_id(0)

        def fire(step, slot):
            t0 = step * TC_BLK
            for u in range(TC_BLK):
                pltpu.make_async_copy(
                    sim_any.at[pl.ds(tgt_s[t0 + u], 1)],
                    sim_buf.at[slot, pl.ds(u, 1)],
                    sems.at[slot],
                ).start()

        def drain(step, slot):
            t0 = step * TC_BLK
            for u in range(TC_BLK):
                pltpu.make_async_copy(
                    sim_any.at[pl.ds(tgt_s[t0 + u], 1)],
                    sim_buf.at[slot, pl.ds(u, 1)],
                    sems.at[slot],
                ).wait()

        @pl.when(j == 0)
        def _():
            out_num[0, 0] = 0.0
            out_den[0, 0] = 0.0
            out_ml[0, 0] = 0.0
            out_m[0, 0] = 0.0
            fire(0, 0)
            if steps > 1:
                fire(1, 1)
            if steps > 2:
                fire(2, 2)

        @pl.when(j + 3 < steps)
        def _():
            fire(j + 3, (j + 3) % 4)

        slot = j % 4
        drain(j, slot)
        sim8 = sim_buf[slot]
        c2 = 1.4426950408889634 / TAU_WORD
        e = jnp.exp2(sim8 * c2 - c2)
        in8 = in_ref[0]
        t0 = j * TC_BLK
        mcol = jnp.stack(
            [msk_s[t0 + u] for u in range(TC_BLK)]).reshape(TC_BLK, 1)
        tcol = jnp.stack(
            [tgt_s[t0 + u] for u in range(TC_BLK)]).reshape(TC_BLK, 1)
        s_num = jnp.sum(in8 * e, axis=1, keepdims=True)
        s_den = jnp.sum(e, axis=1, keepdims=True)
        lane = lax.broadcasted_iota(jnp.int32, (TC_BLK, V), 1)
        g = jnp.sum(jnp.where(lane == tcol, in8, 0.0), axis=1, keepdims=True)
        out_num[0, 0] += jnp.sum(s_num * mcol)
        out_den[0, 0] += jnp.sum(s_den * mcol)
        out_ml[0, 0] += jnp.sum(g * mcol)
        out_m[0, 0] += jnp.sum(mcol)

    in_spec = pl.BlockSpec(
        (1, TC_BLK, V), lambda j, tgt, msk: (j // bb, j % bb, 0))
    sim_spec = pl.BlockSpec(memory_space=pl.ANY)
    scalar_spec = pl.BlockSpec(memory_space=pltpu.SMEM)
    grid_spec = pltpu.PrefetchScalarGridSpec(
        num_scalar_prefetch=2,
        grid=(steps,),
        in_specs=[in_spec, sim_spec],
        out_specs=[scalar_spec] * 4,
        scratch_shapes=[
            pltpu.VMEM((4, TC_BLK, V), jnp.float32),
            pltpu.SemaphoreType.DMA((4,)),
        ],
    )
    return pl.pallas_call(
        body,
        grid_spec=grid_spec,
        out_shape=[jax.ShapeDtypeStruct((1, 1), jnp.float32)] * 4,
        compiler_params=pltpu.CompilerParams(
            dimension_semantics=("arbitrary",)),
    )


def _epilogue_body(p_ref, tn_ref, td_ref, tml_ref, tm_ref, oml_ref, otot_ref):
    p = p_ref[...]
    num = jnp.sum(p[:, 0:L]) + tn_ref[0, 0]
    den = jnp.sum(p[:, L:2 * L]) + td_ref[0, 0]
    ml_sum = jnp.sum(p[:, 2 * L:3 * L]) + tml_ref[0, 0]
    m_sum = jnp.sum(p[:, 3 * L:4 * L]) + tm_ref[0, 0]
    ml_output = -ml_sum / m_sum
    total = ALPHA * (-num / den) + (1.0 - ALPHA) * ml_output
    oml_ref[0, 0] = ml_output
    otot_ref[0, 0] = total


_epilogue = pl.pallas_call(
    _epilogue_body,
    out_shape=[jax.ShapeDtypeStruct((1, 1), jnp.float32)] * 2,
    out_specs=[pl.BlockSpec(memory_space=pltpu.SMEM)] * 2,
)

def kernel(input, target, mask, Sim_Matrix):
    b, t, v = input.shape
    input_t = jnp.swapaxes(input, 0, 1)
    flat_t = jnp.swapaxes(target[:, :t], 0, 1).reshape(-1).astype(jnp.int32)
    n = flat_t.shape[0]
    tgt8 = jnp.pad(flat_t.reshape(n // 2, 2),
                   ((0, 0), (0, 14))).reshape(-1)
    flat_m = jnp.swapaxes(mask[:, :t], 0, 1).reshape(-1).astype(jnp.float32)
    sim_tail = Sim_Matrix[:, v - LANE_TILE:]
    k = TC_TOKENS if (n - TC_TOKENS) % (8 * NW) == 0 and b % TC_BLK == 0 else 0
    partials = _make_sc_call(b, t, v, k)(
        input_t, tgt8, flat_m, Sim_Matrix, sim_tail)
    if k:
        tc_num, tc_den, tc_ml, tc_m = _make_tc_call(b, t, v, k)(
            flat_t, flat_m, input_t, Sim_Matrix)
    else:
        z = jnp.zeros((1, 1), jnp.float32)
        tc_num = tc_den = tc_ml = tc_m = z
    oml, otot = _epilogue(partials, tc_num, tc_den, tc_ml, tc_m)
    return (oml.reshape(()), otot.reshape(()))
